# baseline (speedup 1.0000x reference)
import functools

import jax
import jax.numpy as jnp
from jax import lax
from jax.experimental import pallas as pl
from jax.experimental.pallas import tpu as pltpu
from jax.experimental.pallas import tpu_sc as plsc

VOCAB = 1000000
FEATURES = 64
BATCH = 16384

NC = 2
NS = 16
NW = NC * NS
PAIRS_PER_W = BATCH // NW
ROWS_PER_W = 2 * PAIRS_PER_W
IDX_CHUNK = 128
N_CHUNKS = ROWS_PER_W // IDX_CHUNK


def _sc_kernel(x_hbm, w_hbm, out_hbm, idx_v, rows_v, part_v, dots_v, sem):
    c = lax.axis_index("c")
    s = lax.axis_index("s")
    wid = s * NC + c
    base = wid * PAIRS_PER_W

    pltpu.sync_copy(x_hbm.at[wid], idx_v)

    copies = [
        pltpu.async_copy(
            w_hbm.at[idx_v.at[j]],
            rows_v.at[pl.ds(j * IDX_CHUNK, IDX_CHUNK)],
            sem,
        )
        for j in range(N_CHUNKS)
    ]
    for cp in copies:
        cp.wait()

    def pair_body(i, _):
        r = 2 * i
        acc = rows_v[r, pl.ds(0, 16)] * rows_v[r + 1, pl.ds(0, 16)]
        for k in range(1, 4):
            acc = acc + rows_v[r, pl.ds(16 * k, 16)] * rows_v[r + 1, pl.ds(16 * k, 16)]
        part_v[i, pl.ds(0, 16)] = acc
        return 0

    lax.fori_loop(0, PAIRS_PER_W, pair_body, 0)

    lane = lax.iota(jnp.int32, 16)

    def red_body(g, _):
        rows_idx = 16 * g + lane
        d = plsc.load_gather(part_v, [rows_idx, jnp.zeros((16,), jnp.int32)])
        for l in range(1, 16):
            d = d + plsc.load_gather(part_v, [rows_idx, jnp.full((16,), l, jnp.int32)])
        dots_v[pl.ds(g * 16, 16)] = 1.0 / (1.0 + jnp.exp(-d))
        return 0

    lax.fori_loop(0, PAIRS_PER_W // 16, red_body, 0)

    pltpu.sync_copy(dots_v, out_hbm.at[pl.ds(base, PAIRS_PER_W)])


@jax.jit
def kernel(x, W_g):
    x3 = x.reshape(NW, N_CHUNKS, IDX_CHUNK)
    mesh = plsc.VectorSubcoreMesh(core_axis_name="c", subcore_axis_name="s")
    run = functools.partial(
        pl.kernel,
        mesh=mesh,
        out_type=jax.ShapeDtypeStruct((BATCH,), jnp.float32),
        scratch_types=[
            pltpu.VMEM((N_CHUNKS, IDX_CHUNK), jnp.int32),
            pltpu.VMEM((ROWS_PER_W, FEATURES), jnp.float32),
            pltpu.VMEM((PAIRS_PER_W, 17), jnp.float32),
            pltpu.VMEM((PAIRS_PER_W,), jnp.float32),
            pltpu.SemaphoreType.DMA,
        ],
        compiler_params=pltpu.CompilerParams(
            needs_layout_passes=False, use_tc_tiling_on_sc=False
        ),
    )(_sc_kernel)
    out = run(x3, W_g)
    return out.reshape(BATCH, 1)
